# baseline (speedup 1.0000x reference)
import jax
import jax.numpy as jnp
from jax.experimental import pallas as pl
from jax.experimental.pallas import tpu as pltpu

_V, _B, _C, _H, _W = 4, 8, 5, 512, 512
_HW = _H * _W
_Q = 85.0
_MIN_AREA = 0.01
_IGNORE = -1
_RH = 512
_NJ = _H // _RH
_RH2 = 256
_NJ2 = _H // _RH2


_G = 4


def _count_body(pred_ref, out_ref):
    sub = jax.lax.broadcasted_iota(jnp.int32, (8, 128), 0)
    lane = jax.lax.broadcasted_iota(jnp.int32, (8, 128), 1)
    for g in range(_G):
        x0 = pred_ref[5 * g + 0]
        x1 = pred_ref[5 * g + 1]
        x2 = pred_ref[5 * g + 2]
        x3 = pred_ref[5 * g + 3]
        x4 = pred_ref[5 * g + 4]
        p2 = jnp.maximum(x0, x1)
        p3 = jnp.maximum(p2, x2)
        p4 = jnp.maximum(p3, x3)
        s3 = jnp.maximum(x3, x4)
        s2 = jnp.maximum(x2, s3)
        s1 = jnp.maximum(x1, s2)
        g1 = jnp.sum((s1 > x0).astype(jnp.float32))
        g2 = jnp.sum((s2 > p2).astype(jnp.float32))
        g3 = jnp.sum((s3 > p3).astype(jnp.float32))
        g4 = jnp.sum((x4 > p4).astype(jnp.float32))
        vec = jnp.zeros((8, 128), jnp.float32)
        for k, gk in enumerate((g1, g2, g3, g4)):
            vec = jnp.where((sub == 0) & (lane == k), gk, vec)
        out_ref[g] = vec


def _rewrite_body(anom_ref, lab_ref, out_ref):
    b = pl.program_id(0)
    lab = lab_ref[0]
    safe = jnp.clip(lab, 0, _C - 1)
    a0 = anom_ref[b, 0]
    a1 = anom_ref[b, 1]
    a2 = anom_ref[b, 2]
    a3 = anom_ref[b, 3]
    a4 = anom_ref[b, 4]
    af = jnp.where(
        safe == 0, a0,
        jnp.where(safe == 1, a1,
                  jnp.where(safe == 2, a2,
                            jnp.where(safe == 3, a3, a4))))
    out_ref[0] = jnp.where((lab != _IGNORE) & (af > 0), _IGNORE, lab)


def _quantile_thr(scores, mask):
    q = _Q / 100.0
    flat = jnp.where(mask, scores, jnp.inf).reshape(-1)
    s = jnp.sort(flat)
    n = jnp.sum(mask)
    pos = q * jnp.maximum(n - 1, 0).astype(jnp.float32)
    lo = jnp.floor(pos).astype(jnp.int32)
    hi = jnp.ceil(pos).astype(jnp.int32)
    frac = pos - lo.astype(jnp.float32)
    val = s[lo] * (1.0 - frac) + s[hi] * frac
    return jnp.where(n > 0, val, jnp.inf)


def kernel(predictions_list, pseudo_labels_to_modify):
    preds = predictions_list.reshape(_V * _B * _C, _H, _W)
    counts_s = pl.pallas_call(
        _count_body,
        grid=(_V * _B // _G,),
        in_specs=[pl.BlockSpec((_C * _G, _H, _W), lambda i: (i, 0, 0))],
        out_specs=pl.BlockSpec((_G, 8, 128), lambda i: (i, 0, 0)),
        out_shape=jax.ShapeDtypeStruct((_V * _B, 8, 128), jnp.float32),
    )(preds)
    s = counts_s[:, 0, :4].reshape(_V, _B, 4)
    c0 = _HW - s[..., 0]
    c1 = s[..., 0] - s[..., 1]
    c2 = s[..., 1] - s[..., 2]
    c3 = s[..., 2] - s[..., 3]
    c4 = s[..., 3]
    stacked = jnp.stack([c0, c1, c2, c3, c4], axis=2)
    stacked = stacked.transpose(1, 2, 0)
    mean_c = jnp.mean(stacked, axis=2)
    std_c = jnp.std(stacked, axis=2, ddof=1)
    scores = std_c / (mean_c + 1e-08)
    scores = jnp.where(mean_c == 0, 0.0, scores)
    sig = (mean_c / _HW) > _MIN_AREA
    sig = sig.at[:, 0].set(False)
    thr = _quantile_thr(scores, sig)
    is_anom = ((scores > thr) & sig).astype(jnp.int32)
    final = pl.pallas_call(
        _rewrite_body,
        grid=(_B, _NJ2),
        in_specs=[
            pl.BlockSpec(memory_space=pltpu.SMEM),
            pl.BlockSpec((1, _RH2, _W), lambda b, j: (b, j, 0)),
        ],
        out_specs=pl.BlockSpec((1, _RH2, _W), lambda b, j: (b, j, 0)),
        out_shape=jax.ShapeDtypeStruct((_B, _H, _W), jnp.int32),
    )(is_anom, pseudo_labels_to_modify)
    return final


# speedup vs baseline: 1.1290x; 1.1290x over previous
import jax
import jax.numpy as jnp
from jax.experimental import pallas as pl
from jax.experimental.pallas import tpu as pltpu

_V, _B, _C, _H, _W = 4, 8, 5, 512, 512
_HW = _H * _W
_Q = 85.0
_MIN_AREA = 0.01
_IGNORE = -1
_G = 2
_NC = _V * _B // _G
_GB = 4
_NR = _B // _GB


def _body(pred_ref, lab_ref, out_ref, cnt_ref, flag_ref):
    step = pl.program_id(0)
    sub = jax.lax.broadcasted_iota(jnp.int32, (8, 128), 0)
    lane = jax.lax.broadcasted_iota(jnp.int32, (8, 128), 1)

    @pl.when(step < _NC)
    def _count():
        for g in range(_G):
            x0 = pred_ref[5 * g + 0]
            x1 = pred_ref[5 * g + 1]
            x2 = pred_ref[5 * g + 2]
            x3 = pred_ref[5 * g + 3]
            x4 = pred_ref[5 * g + 4]
            p2 = jnp.maximum(x0, x1)
            p3 = jnp.maximum(p2, x2)
            p4 = jnp.maximum(p3, x3)
            s3 = jnp.maximum(x3, x4)
            s2 = jnp.maximum(x2, s3)
            s1 = jnp.maximum(x1, s2)
            g1 = jnp.sum((s1 > x0).astype(jnp.float32))
            g2 = jnp.sum((s2 > p2).astype(jnp.float32))
            g3 = jnp.sum((s3 > p3).astype(jnp.float32))
            g4 = jnp.sum((x4 > p4).astype(jnp.float32))
            vec = jnp.zeros((8, 128), jnp.float32)
            for k, gk in enumerate((g1, g2, g3, g4)):
                vec = jnp.where((sub == 0) & (lane == k), gk, vec)
            cnt_ref[pl.ds(_G * step + g, 1)] = vec[None]

    @pl.when(step == _NC - 1)
    def _stats():
        cnts = []
        for v in range(_V):
            vb = jnp.zeros((8, 128), jnp.float32)
            for b in range(_B):
                vb = vb + pltpu.roll(cnt_ref[v * _B + b], b, 0)
            r = pltpu.roll(vb, 1, 1)
            cnt_v = jnp.where(lane == 0, float(_HW) - vb, r - vb)
            cnt_v = jnp.where(lane < _C, cnt_v, 0.0)
            cnts.append(cnt_v)
        mean = (cnts[0] + cnts[1] + cnts[2] + cnts[3]) * 0.25
        d0 = cnts[0] - mean
        d1 = cnts[1] - mean
        d2 = cnts[2] - mean
        d3 = cnts[3] - mean
        var = (((d0 * d0 + d1 * d1) + d2 * d2) + d3 * d3) * (1.0 / 3.0)
        scores = jnp.where(mean == 0.0, 0.0,
                           jnp.sqrt(var) / (mean + 1e-08))
        sig = (mean * (1.0 / _HW) > _MIN_AREA) & (lane >= 1) & (lane < _C)
        vals = jnp.where(sig, scores, jnp.float32(jnp.inf))
        posv = sub * _C + lane
        n = jnp.sum(sig.astype(jnp.float32))
        pos = (_Q / 100.0) * jnp.maximum(n - 1.0, 0.0)
        lo = jnp.floor(pos)
        hi = jnp.ceil(pos)
        frac = pos - lo
        big = jnp.int32(1 << 30)

        def _knock(vv):
            m = jnp.min(vv)
            p = jnp.min(jnp.where(vv == m, posv, big))
            return m, jnp.where(posv == p, jnp.float32(jnp.inf), vv)

        vals_lo = jax.lax.fori_loop(
            0, lo.astype(jnp.int32), lambda k, vv: _knock(vv)[1], vals)
        s_lo, vals_hi = _knock(vals_lo)
        s_hi = jnp.where(hi > lo, jnp.min(vals_hi), s_lo)
        val = s_lo * (1.0 - frac) + s_hi * frac
        thr = jnp.where(n > 0, val, jnp.float32(jnp.inf))
        an = ((scores > thr) & sig).astype(jnp.int32)
        for b in range(_B):
            for c in range(_C):
                flag_ref[b, c] = jnp.sum(
                    jnp.where((sub == b) & (lane == c), an, 0))

    @pl.when(step >= _NC)
    def _rewrite():
        for q in range(_GB):
            b = (step - _NC) * _GB + q
            lab = lab_ref[q]
            safe = jnp.clip(lab, 0, _C - 1)
            a0 = flag_ref[b, 0]
            a1 = flag_ref[b, 1]
            a2 = flag_ref[b, 2]
            a3 = flag_ref[b, 3]
            a4 = flag_ref[b, 4]
            af = jnp.where(
                safe == 0, a0,
                jnp.where(safe == 1, a1,
                          jnp.where(safe == 2, a2,
                                    jnp.where(safe == 3, a3, a4))))
            out_ref[q] = jnp.where((lab != _IGNORE) & (af > 0), _IGNORE, lab)


def kernel(predictions_list, pseudo_labels_to_modify):
    preds = predictions_list.reshape(_V * _B * _C, _H, _W)
    final = pl.pallas_call(
        _body,
        grid=(_NC + _NR,),
        in_specs=[
            pl.BlockSpec((_C * _G, _H, _W),
                         lambda s: (jnp.minimum(s, _NC - 1), 0, 0)),
            pl.BlockSpec((_GB, _H, _W),
                         lambda s: (jnp.maximum(s - _NC, 0), 0, 0)),
        ],
        out_specs=pl.BlockSpec((_GB, _H, _W),
                               lambda s: (jnp.maximum(s - _NC, 0), 0, 0)),
        out_shape=jax.ShapeDtypeStruct((_B, _H, _W), jnp.int32),
        scratch_shapes=[
            pltpu.VMEM((_V * _B, 8, 128), jnp.float32),
            pltpu.SMEM((_B, _C), jnp.int32),
        ],
    )(preds, pseudo_labels_to_modify)
    return final


# speedup vs baseline: 1.3036x; 1.1546x over previous
import jax
import jax.numpy as jnp
from jax.experimental import pallas as pl
from jax.experimental.pallas import tpu as pltpu

_V, _B, _C, _H, _W = 4, 8, 5, 512, 512
_HW = _H * _W
_Q = 85.0
_MIN_AREA = 0.01
_IGNORE = -1
_G = 2
_NC = _V * _B // _G
_GB = 4
_NR = _B // _GB


def _body(pred_ref, lab_ref, out_ref, cnt_ref, flag_ref):
    step = pl.program_id(0)
    sub = jax.lax.broadcasted_iota(jnp.int32, (8, 128), 0)
    lane = jax.lax.broadcasted_iota(jnp.int32, (8, 128), 1)

    @pl.when(step < _NC)
    def _count():
        for g in range(_G):
            x0 = pred_ref[5 * g + 0]
            x1 = pred_ref[5 * g + 1]
            x2 = pred_ref[5 * g + 2]
            x3 = pred_ref[5 * g + 3]
            x4 = pred_ref[5 * g + 4]
            p2 = jnp.maximum(x0, x1)
            p3 = jnp.maximum(p2, x2)
            p4 = jnp.maximum(p3, x3)
            s3 = jnp.maximum(x3, x4)
            s2 = jnp.maximum(x2, s3)
            s1 = jnp.maximum(x1, s2)
            g1 = jnp.sum((s1 > x0).astype(jnp.float32))
            g2 = jnp.sum((s2 > p2).astype(jnp.float32))
            g3 = jnp.sum((s3 > p3).astype(jnp.float32))
            g4 = jnp.sum((x4 > p4).astype(jnp.float32))
            vec = jnp.zeros((8, 128), jnp.float32)
            for k, gk in enumerate((g1, g2, g3, g4)):
                vec = jnp.where((sub == 0) & (lane == k), gk, vec)
            cnt_ref[pl.ds(_G * step + g, 1)] = vec[None]

    @pl.when(step == _NC - 1)
    def _stats():
        cnts = []
        for v in range(_V):
            vb = jnp.zeros((8, 128), jnp.float32)
            for b in range(_B):
                vb = vb + pltpu.roll(cnt_ref[v * _B + b], b, 0)
            r = pltpu.roll(vb, 1, 1)
            cnt_v = jnp.where(lane == 0, float(_HW) - vb, r - vb)
            cnt_v = jnp.where(lane < _C, cnt_v, 0.0)
            cnts.append(cnt_v)
        mean = (cnts[0] + cnts[1] + cnts[2] + cnts[3]) * 0.25
        d0 = cnts[0] - mean
        d1 = cnts[1] - mean
        d2 = cnts[2] - mean
        d3 = cnts[3] - mean
        var = (((d0 * d0 + d1 * d1) + d2 * d2) + d3 * d3) * (1.0 / 3.0)
        scores = jnp.where(mean == 0.0, 0.0,
                           jnp.sqrt(var) / (mean + 1e-08))
        sig = (mean * (1.0 / _HW) > _MIN_AREA) & (lane >= 1) & (lane < _C)
        vals = jnp.where(sig, scores, jnp.float32(-jnp.inf))
        posv = sub * _C + lane
        n = jnp.sum(sig.astype(jnp.float32))
        pos = (_Q / 100.0) * jnp.maximum(n - 1.0, 0.0)
        lo = jnp.floor(pos)
        hi = jnp.ceil(pos)
        frac = pos - lo
        big = jnp.int32(1 << 30)

        def _knock(vv):
            m = jnp.max(vv)
            p = jnp.min(jnp.where(vv == m, posv, big))
            return m, jnp.where(posv == p, jnp.float32(-jnp.inf), vv)

        t = (n - 1.0 - hi).astype(jnp.int32)
        vals_t = jax.lax.fori_loop(
            0, t, lambda k, vv: _knock(vv)[1], vals)
        s_hi, vals_r = _knock(vals_t)
        s_lo = jnp.where(hi > lo, jnp.max(vals_r), s_hi)
        val = s_lo * (1.0 - frac) + s_hi * frac
        thr = jnp.where(n > 0, val, jnp.float32(jnp.inf))
        an = ((scores > thr) & sig).astype(jnp.int32)
        sh = 5 * jnp.where(sub < 4, sub, sub - 4) + lane
        w = jnp.where(lane < _C, jnp.left_shift(1, sh), 0)
        m_lo = jnp.sum(jnp.where(sub < 4, an * w, 0))
        m_hi = jnp.sum(jnp.where(sub >= 4, an * w, 0))
        for b in range(_B):
            src = m_lo if b < 4 else m_hi
            flag_ref[b] = jnp.right_shift(src, 5 * (b % 4)) & 31

    @pl.when(step >= _NC)
    def _rewrite():
        for q in range(_GB):
            b = (step - _NC) * _GB + q
            lab = lab_ref[q]
            safe = jnp.clip(lab, 0, _C - 1)
            fl = jnp.right_shift(flag_ref[b], safe) & 1
            out_ref[q] = jnp.where((lab != _IGNORE) & (fl > 0), _IGNORE, lab)


def kernel(predictions_list, pseudo_labels_to_modify):
    preds = predictions_list.reshape(_V * _B * _C, _H, _W)
    final = pl.pallas_call(
        _body,
        grid=(_NC + _NR,),
        in_specs=[
            pl.BlockSpec((_C * _G, _H, _W),
                         lambda s: (jnp.minimum(s, _NC - 1), 0, 0)),
            pl.BlockSpec((_GB, _H, _W),
                         lambda s: (jnp.maximum(s - _NC, 0), 0, 0)),
        ],
        out_specs=pl.BlockSpec((_GB, _H, _W),
                               lambda s: (jnp.maximum(s - _NC, 0), 0, 0)),
        out_shape=jax.ShapeDtypeStruct((_B, _H, _W), jnp.int32),
        scratch_shapes=[
            pltpu.VMEM((_V * _B, 8, 128), jnp.float32),
            pltpu.SMEM((_B,), jnp.int32),
        ],
    )(preds, pseudo_labels_to_modify)
    return final
